# baseline (speedup 1.0000x reference)
import functools

import jax
import jax.numpy as jnp
import numpy as np
from jax import lax
from jax.experimental import pallas as pl
from jax.experimental.pallas import tpu as pltpu
from jax.experimental.pallas import tpu_sc as plsc

B = 16384
F = 26
D = 32
BF = B * F
CARD = 100000
V = F * CARD
NSR = V // 4

NC = 2
NS = 16
NW = NC * NS
PER_W = BF // NW

CH = 416
NCHUNK = PER_W // CH
NSTREAM = 4
GPC = CH // 16

NPAN = 20313
PFULL = 20312
PAN_PER_W = 635

_cparams = pltpu.CompilerParams(
    use_tc_tiling_on_sc=True, needs_layout_passes=False)


def _transpose_panel(pan_v, srow_v, nrow):
    for t in range(nrow):
        for v in range(8):
            dvec = lax.iota(jnp.int32, 16) + (v % 2) * 16
            lvec = jnp.full((16,), 4 * t + v // 2, jnp.int32)
            srow_v[t, pl.ds(v * 16, 16)] = plsc.load_gather(pan_v, [dvec, lvec])


def _body_a(tabT_hbm, tail_hbm, tr_hbm, pan_v, srow_v):
    wid = lax.axis_index("s") * NC + lax.axis_index("c")
    p0 = wid * PAN_PER_W

    def panel(p, _):
        pltpu.sync_copy(
            tabT_hbm.at[:, pl.ds(pl.multiple_of(p * 128, 128), 128)], pan_v)
        _transpose_panel(pan_v, srow_v, 32)
        pltpu.sync_copy(srow_v, tr_hbm.at[pl.ds(pl.multiple_of(p * 32, 32), 32)])
        return _

    phi = jnp.minimum((wid + 1) * PAN_PER_W, PFULL)
    lax.fori_loop(p0, phi, panel, 0)

    @pl.when(wid == NW - 1)
    def _():
        pltpu.sync_copy(tail_hbm, tr_hbm.at[pl.ds(PFULL * 32, 16)])


def _body_b(tr_hbm, x_hbm, sh_hbm, off_hbm, out_hbm,
            idx_v, sidx_v, qoff_v, srows_v, o_v, sh_v, shx_v, off_v, sem):
    wid = lax.axis_index("s") * NC + lax.axis_index("c")
    base = wid * PER_W

    pltpu.sync_copy(sh_hbm, sh_v)
    pltpu.sync_copy(off_hbm, off_v)

    iota = lax.iota(jnp.int32, 16)
    r2pat = lax.shift_right_logical(iota, 2)
    c2pat = lax.shift_left(jnp.bitwise_and(iota, 3), 5)

    for phase in range(13):
        fv = iota + (phase * 16) % F
        fv = jnp.where(fv >= F, fv - F, fv)
        fb = fv * D
        for dd in range(D):
            shx_v[pl.ds((phase * D + dd) * 16, 16)] = plsc.load_gather(
                sh_v, [fb + dd])

    def chunk(c, _):
        start = pl.multiple_of(base + c * CH, CH)
        pltpu.sync_copy(x_hbm.at[pl.ds(start, CH)], idx_v)
        for k in range(GPC):
            sl = pl.ds(k * 16, 16)
            t = idx_v[sl] + off_v[sl]
            sidx_v[sl] = lax.shift_right_logical(t, 2)
            qoff_v[sl] = lax.shift_left(jnp.bitwise_and(t, 3), 5)

        copies = []
        for j in range(NSTREAM):
            copies.append(pltpu.async_copy(
                tr_hbm.at[sidx_v.at[pl.ds(j * 104, 104)]],
                srows_v.at[pl.ds(j * 104, 104)],
                sem))
        for cp in copies:
            cp.wait()

        def half(h, _):
            rb = h * 208
            for g in range(13):
                rowbase = rb + g * 16
                rvec = rowbase + iota
                qv = plsc.load_gather(qoff_v, [rvec])
                r2 = (rowbase // 4) + r2pat
                for dd in range(D):
                    val = plsc.load_gather(srows_v, [rvec, qv + dd])
                    val = val + shx_v[pl.ds((g * D + dd) * 16, 16)]
                    plsc.store_scatter(o_v, [r2, c2pat + dd], val)
            return _
        lax.fori_loop(0, 2, half, 0)

        pltpu.sync_copy(
            o_v, out_hbm.at[pl.ds(pl.multiple_of(start // 4, CH // 4), CH // 4)])
        return _

    lax.fori_loop(0, NCHUNK, chunk, 0)


def kernel(x, indiv_embed, shared_embed):
    offsets = np.arange(F, dtype=np.int32) * CARD
    offpat = jnp.asarray(np.tile(offsets, CH // F))

    mesh = plsc.VectorSubcoreMesh(core_axis_name="c", subcore_axis_name="s")

    run_a = pl.kernel(
        _body_a,
        out_type=jax.ShapeDtypeStruct((NSR, 128), jnp.float32),
        mesh=mesh,
        scratch_types=[
            pltpu.VMEM((32, 128), jnp.float32),
            pltpu.VMEM((32, 128), jnp.float32),
        ],
        compiler_params=_cparams,
    )

    run_b = pl.kernel(
        _body_b,
        out_type=jax.ShapeDtypeStruct((BF // 4, 128), jnp.float32),
        mesh=mesh,
        scratch_types=[
            pltpu.VMEM((CH,), jnp.int32),
            pltpu.VMEM((CH,), jnp.int32),
            pltpu.VMEM((CH,), jnp.int32),
            pltpu.VMEM((CH, 128), jnp.float32),
            pltpu.VMEM((CH // 4, 128), jnp.float32),
            pltpu.VMEM((F * D,), jnp.float32),
            pltpu.VMEM((13 * D * 16,), jnp.float32),
            pltpu.VMEM((CH,), jnp.int32),
            pltpu.SemaphoreType.DMA,
        ],
        compiler_params=_cparams,
    )

    tail = indiv_embed[V - 64:].reshape(16, 128)
    tr = run_a(indiv_embed.T, tail)
    out = run_b(tr, x.reshape(BF), shared_embed.reshape(F * D), offpat)
    return out.reshape(BF, D).reshape(B, F, D)


# speedup vs baseline: 1.2242x; 1.2242x over previous
import functools

import jax
import jax.numpy as jnp
import numpy as np
from jax import lax
from jax.experimental import pallas as pl
from jax.experimental.pallas import tpu as pltpu
from jax.experimental.pallas import tpu_sc as plsc

B = 16384
F = 26
D = 32
BF = B * F
CARD = 100000
V = F * CARD
NSR = V // 4

NC = 2
NS = 16
NW = NC * NS
PER_W = BF // NW

CH = 416
NCHUNK = PER_W // CH
NSTREAM = 4
GPC = CH // 16

NPAN = 20313
PFULL = 20312
PAN_PER_W = 635

_cparams = pltpu.CompilerParams(
    use_tc_tiling_on_sc=True, needs_layout_passes=False)


def _transpose_panel(pan_v, srow_v, nrow):
    for t in range(nrow):
        for v in range(8):
            dvec = lax.iota(jnp.int32, 16) + (v % 2) * 16
            lvec = jnp.full((16,), 4 * t + v // 2, jnp.int32)
            srow_v[t, pl.ds(v * 16, 16)] = plsc.load_gather(pan_v, [dvec, lvec])


def _body_a(tabT_hbm, tail_hbm, tr_hbm,
            pan0_v, pan1_v, srow0_v, srow1_v, isem, osem):
    wid = lax.axis_index("s") * NC + lax.axis_index("c")
    p0 = wid * PAN_PER_W
    phi = jnp.minimum((wid + 1) * PAN_PER_W, PFULL)
    n = phi - p0
    pans = (pan0_v, pan1_v)
    srows = (srow0_v, srow1_v)

    def start_in(p, b):
        pltpu.async_copy(
            tabT_hbm.at[:, pl.ds(pl.multiple_of(p * 128, 128), 128)],
            pans[b], isem)

    def start_out(p, b):
        pltpu.async_copy(
            srows[b], tr_hbm.at[pl.ds(pl.multiple_of(p * 32, 32), 32)], osem)

    def drain_in(b):
        pltpu.make_async_copy(
            tabT_hbm.at[:, pl.ds(0, 128)], pans[b], isem).wait()

    def drain_out(b):
        pltpu.make_async_copy(
            srows[b], tr_hbm.at[pl.ds(0, 32)], osem).wait()

    @pl.when(n > 0)
    def _():
        start_in(p0, 0)
    @pl.when(n > 1)
    def _():
        start_in(p0 + 1, 1)

    def step(i, carry):
        for b in range(2):
            p = p0 + i * 2 + b

            @pl.when(p < phi)
            def _body():
                drain_in(b)
                @pl.when(i > 0)
                def _reclaim():
                    drain_out(b)
                _transpose_panel(pans[b], srows[b], 32)
                start_out(p, b)

                @pl.when(p + 2 < phi)
                def _next():
                    start_in(p + 2, b)
        return carry

    lax.fori_loop(0, (n + 1) // 2, step, 0)

    @pl.when(n > 0)
    def _d0():
        drain_out(0)
    @pl.when(n > 1)
    def _d1():
        drain_out(1)

    @pl.when(wid == NW - 1)
    def _():
        pltpu.sync_copy(tail_hbm, tr_hbm.at[pl.ds(PFULL * 32, 16)])


def _body_b(tr_hbm, x_hbm, sh_hbm, off_hbm, out_hbm,
            idx_v, sidx_v, qoff_v, srows_v, o_v, sh_v, shx_v, off_v, sem):
    wid = lax.axis_index("s") * NC + lax.axis_index("c")
    base = wid * PER_W

    pltpu.sync_copy(sh_hbm, sh_v)
    pltpu.sync_copy(off_hbm, off_v)

    iota = lax.iota(jnp.int32, 16)
    r2pat = lax.shift_right_logical(iota, 2)
    c2pat = lax.shift_left(jnp.bitwise_and(iota, 3), 5)

    for phase in range(13):
        fv = iota + (phase * 16) % F
        fv = jnp.where(fv >= F, fv - F, fv)
        fb = fv * D
        for dd in range(D):
            shx_v[pl.ds((phase * D + dd) * 16, 16)] = plsc.load_gather(
                sh_v, [fb + dd])

    def chunk(c, _):
        start = pl.multiple_of(base + c * CH, CH)
        pltpu.sync_copy(x_hbm.at[pl.ds(start, CH)], idx_v)
        for k in range(GPC):
            sl = pl.ds(k * 16, 16)
            t = idx_v[sl] + off_v[sl]
            sidx_v[sl] = lax.shift_right_logical(t, 2)
            qoff_v[sl] = lax.shift_left(jnp.bitwise_and(t, 3), 5)

        copies = []
        for j in range(NSTREAM):
            copies.append(pltpu.async_copy(
                tr_hbm.at[sidx_v.at[pl.ds(j * 104, 104)]],
                srows_v.at[pl.ds(j * 104, 104)],
                sem))
        for cp in copies:
            cp.wait()

        def half(h, _):
            rb = h * 208
            for g in range(13):
                rowbase = rb + g * 16
                rvec = rowbase + iota
                qv = plsc.load_gather(qoff_v, [rvec])
                r2 = (rowbase // 4) + r2pat
                for dd in range(D):
                    val = plsc.load_gather(srows_v, [rvec, qv + dd])
                    val = val + shx_v[pl.ds((g * D + dd) * 16, 16)]
                    plsc.store_scatter(o_v, [r2, c2pat + dd], val)
            return _
        lax.fori_loop(0, 2, half, 0)

        pltpu.sync_copy(
            o_v, out_hbm.at[pl.ds(pl.multiple_of(start // 4, CH // 4), CH // 4)])
        return _

    lax.fori_loop(0, NCHUNK, chunk, 0)


def kernel(x, indiv_embed, shared_embed):
    offsets = np.arange(F, dtype=np.int32) * CARD
    offpat = jnp.asarray(np.tile(offsets, CH // F))

    mesh = plsc.VectorSubcoreMesh(core_axis_name="c", subcore_axis_name="s")

    run_a = pl.kernel(
        _body_a,
        out_type=jax.ShapeDtypeStruct((NSR, 128), jnp.float32),
        mesh=mesh,
        scratch_types=[
            pltpu.VMEM((32, 128), jnp.float32),
            pltpu.VMEM((32, 128), jnp.float32),
            pltpu.VMEM((32, 128), jnp.float32),
            pltpu.VMEM((32, 128), jnp.float32),
            pltpu.SemaphoreType.DMA,
            pltpu.SemaphoreType.DMA,
        ],
        compiler_params=_cparams,
    )

    run_b = pl.kernel(
        _body_b,
        out_type=jax.ShapeDtypeStruct((BF // 4, 128), jnp.float32),
        mesh=mesh,
        scratch_types=[
            pltpu.VMEM((CH,), jnp.int32),
            pltpu.VMEM((CH,), jnp.int32),
            pltpu.VMEM((CH,), jnp.int32),
            pltpu.VMEM((CH, 128), jnp.float32),
            pltpu.VMEM((CH // 4, 128), jnp.float32),
            pltpu.VMEM((F * D,), jnp.float32),
            pltpu.VMEM((13 * D * 16,), jnp.float32),
            pltpu.VMEM((CH,), jnp.int32),
            pltpu.SemaphoreType.DMA,
        ],
        compiler_params=_cparams,
    )

    tail = indiv_embed[V - 64:].reshape(16, 128)
    tr = run_a(indiv_embed.T, tail)
    out = run_b(tr, x.reshape(BF), shared_embed.reshape(F * D), offpat)
    return out.reshape(BF, D).reshape(B, F, D)


# speedup vs baseline: 1.4611x; 1.1935x over previous
import functools

import jax
import jax.numpy as jnp
import numpy as np
from jax import lax
from jax.experimental import pallas as pl
from jax.experimental.pallas import tpu as pltpu
from jax.experimental.pallas import tpu_sc as plsc

B = 16384
F = 26
D = 32
BF = B * F
CARD = 100000
V = F * CARD
NSR = V // 4

NC = 2
NS = 16
NW = NC * NS
PER_W = BF // NW

CH = 416
NCHUNK = PER_W // CH
NSTREAM = 4
GPC = CH // 16

NPAN = 20313
PFULL = 20312
PAN_PER_W = 635

_cparams = pltpu.CompilerParams(
    use_tc_tiling_on_sc=True, needs_layout_passes=False)


def _transpose_panel(pan_v, srow_v, nrow):
    iota = lax.iota(jnp.int32, 16)
    rpat = lax.shift_right_logical(iota, 2)
    cpat = lax.shift_left(jnp.bitwise_and(iota, 3), 5)
    pending = []
    for d in range(32):
        for l0 in range(0, 4 * nrow, 16):
            vals = pan_v[d, pl.ds(l0, 16)]
            pending.append((vals, rpat + (l0 // 4), cpat + d))
            if len(pending) > 4:
                v, r, c = pending.pop(0)
                plsc.store_scatter(srow_v, [r, c], v)
    for v, r, c in pending:
        plsc.store_scatter(srow_v, [r, c], v)


def _body_a(tabT_hbm, tail_hbm, tr_hbm,
            pan0_v, pan1_v, srow0_v, srow1_v, isem, osem):
    wid = lax.axis_index("s") * NC + lax.axis_index("c")
    p0 = wid * PAN_PER_W
    phi = jnp.minimum((wid + 1) * PAN_PER_W, PFULL)
    n = phi - p0
    pans = (pan0_v, pan1_v)
    srows = (srow0_v, srow1_v)

    def start_in(p, b):
        pltpu.async_copy(
            tabT_hbm.at[:, pl.ds(pl.multiple_of(p * 128, 128), 128)],
            pans[b], isem)

    def start_out(p, b):
        pltpu.async_copy(
            srows[b], tr_hbm.at[pl.ds(pl.multiple_of(p * 32, 32), 32)], osem)

    def drain_in(b):
        pltpu.make_async_copy(
            tabT_hbm.at[:, pl.ds(0, 128)], pans[b], isem).wait()

    def drain_out(b):
        pltpu.make_async_copy(
            srows[b], tr_hbm.at[pl.ds(0, 32)], osem).wait()

    @pl.when(n > 0)
    def _():
        start_in(p0, 0)
    @pl.when(n > 1)
    def _():
        start_in(p0 + 1, 1)

    def step(i, carry):
        for b in range(2):
            p = p0 + i * 2 + b

            @pl.when(p < phi)
            def _body():
                drain_in(b)
                @pl.when(i > 0)
                def _reclaim():
                    drain_out(b)
                _transpose_panel(pans[b], srows[b], 32)
                start_out(p, b)

                @pl.when(p + 2 < phi)
                def _next():
                    start_in(p + 2, b)
        return carry

    lax.fori_loop(0, (n + 1) // 2, step, 0)

    @pl.when(n > 0)
    def _d0():
        drain_out(0)
    @pl.when(n > 1)
    def _d1():
        drain_out(1)

    @pl.when(wid == NW - 1)
    def _():
        pltpu.sync_copy(tail_hbm, tr_hbm.at[pl.ds(PFULL * 32, 16)])


def _body_b(tr_hbm, x_hbm, sh_hbm, off_hbm, out_hbm,
            idx_v, sidx_v, qoff_v, srows_v, o_v, sh_v, shx_v, off_v, sem):
    wid = lax.axis_index("s") * NC + lax.axis_index("c")
    base = wid * PER_W

    pltpu.sync_copy(sh_hbm, sh_v)
    pltpu.sync_copy(off_hbm, off_v)

    iota = lax.iota(jnp.int32, 16)
    r2pat = lax.shift_right_logical(iota, 2)
    c2pat = lax.shift_left(jnp.bitwise_and(iota, 3), 5)

    for phase in range(13):
        fv = iota + (phase * 16) % F
        fv = jnp.where(fv >= F, fv - F, fv)
        fb = fv * D
        for dd in range(D):
            shx_v[pl.ds((phase * D + dd) * 16, 16)] = plsc.load_gather(
                sh_v, [fb + dd])

    def chunk(c, _):
        start = pl.multiple_of(base + c * CH, CH)
        pltpu.sync_copy(x_hbm.at[pl.ds(start, CH)], idx_v)
        for k in range(GPC):
            sl = pl.ds(k * 16, 16)
            t = idx_v[sl] + off_v[sl]
            sidx_v[sl] = lax.shift_right_logical(t, 2)
            qoff_v[sl] = lax.shift_left(jnp.bitwise_and(t, 3), 5)

        copies = []
        for j in range(NSTREAM):
            copies.append(pltpu.async_copy(
                tr_hbm.at[sidx_v.at[pl.ds(j * 104, 104)]],
                srows_v.at[pl.ds(j * 104, 104)],
                sem))
        for cp in copies:
            cp.wait()

        def half(h, _):
            rb = h * 208
            for g in range(13):
                rowbase = rb + g * 16
                rvec = rowbase + iota
                qv = plsc.load_gather(qoff_v, [rvec])
                r2 = (rowbase // 4) + r2pat
                for dd in range(D):
                    val = plsc.load_gather(srows_v, [rvec, qv + dd])
                    val = val + shx_v[pl.ds((g * D + dd) * 16, 16)]
                    plsc.store_scatter(o_v, [r2, c2pat + dd], val)
            return _
        lax.fori_loop(0, 2, half, 0)

        pltpu.sync_copy(
            o_v, out_hbm.at[pl.ds(pl.multiple_of(start // 4, CH // 4), CH // 4)])
        return _

    lax.fori_loop(0, NCHUNK, chunk, 0)


def kernel(x, indiv_embed, shared_embed):
    offsets = np.arange(F, dtype=np.int32) * CARD
    offpat = jnp.asarray(np.tile(offsets, CH // F))

    mesh = plsc.VectorSubcoreMesh(core_axis_name="c", subcore_axis_name="s")

    run_a = pl.kernel(
        _body_a,
        out_type=jax.ShapeDtypeStruct((NSR, 128), jnp.float32),
        mesh=mesh,
        scratch_types=[
            pltpu.VMEM((32, 128), jnp.float32),
            pltpu.VMEM((32, 128), jnp.float32),
            pltpu.VMEM((32, 128), jnp.float32),
            pltpu.VMEM((32, 128), jnp.float32),
            pltpu.SemaphoreType.DMA,
            pltpu.SemaphoreType.DMA,
        ],
        compiler_params=_cparams,
    )

    run_b = pl.kernel(
        _body_b,
        out_type=jax.ShapeDtypeStruct((BF // 4, 128), jnp.float32),
        mesh=mesh,
        scratch_types=[
            pltpu.VMEM((CH,), jnp.int32),
            pltpu.VMEM((CH,), jnp.int32),
            pltpu.VMEM((CH,), jnp.int32),
            pltpu.VMEM((CH, 128), jnp.float32),
            pltpu.VMEM((CH // 4, 128), jnp.float32),
            pltpu.VMEM((F * D,), jnp.float32),
            pltpu.VMEM((13 * D * 16,), jnp.float32),
            pltpu.VMEM((CH,), jnp.int32),
            pltpu.SemaphoreType.DMA,
        ],
        compiler_params=_cparams,
    )

    tail = indiv_embed[V - 64:].reshape(16, 128)
    tr = run_a(indiv_embed.T, tail)
    out = run_b(tr, x.reshape(BF), shared_embed.reshape(F * D), offpat)
    return out.reshape(BF, D).reshape(B, F, D)


# speedup vs baseline: 1.4612x; 1.0001x over previous
import functools

import jax
import jax.numpy as jnp
import numpy as np
from jax import lax
from jax.experimental import pallas as pl
from jax.experimental.pallas import tpu as pltpu
from jax.experimental.pallas import tpu_sc as plsc

B = 16384
F = 26
D = 32
BF = B * F
CARD = 100000
V = F * CARD
NSR = V // 4

NC = 2
NS = 16
NW = NC * NS
PER_W = BF // NW

CH = 416
NCHUNK = PER_W // CH
NSTREAM = 4
GPC = CH // 16

NPAN = 20313
PFULL = 20312
PAN_PER_W = 635

_cparams = pltpu.CompilerParams(
    use_tc_tiling_on_sc=True, needs_layout_passes=False)


def _transpose_panel(pan_v, srow_v, nrow):
    iota = lax.iota(jnp.int32, 16)
    rpat = lax.shift_right_logical(iota, 2)
    cpat = lax.shift_left(jnp.bitwise_and(iota, 3), 5)
    pending = []
    for d in range(32):
        for l0 in range(0, 4 * nrow, 16):
            vals = pan_v[d, pl.ds(l0, 16)]
            pending.append((vals, rpat + (l0 // 4), cpat + d))
            if len(pending) > 4:
                v, r, c = pending.pop(0)
                plsc.store_scatter(srow_v, [r, c], v)
    for v, r, c in pending:
        plsc.store_scatter(srow_v, [r, c], v)


def _body_a(tabT_hbm, tail_hbm, tr_hbm,
            pan0_v, pan1_v, srow0_v, srow1_v, isem, osem):
    wid = lax.axis_index("s") * NC + lax.axis_index("c")
    p0 = wid * PAN_PER_W
    phi = jnp.minimum((wid + 1) * PAN_PER_W, PFULL)
    n = phi - p0
    pans = (pan0_v, pan1_v)
    srows = (srow0_v, srow1_v)

    def start_in(p, b):
        pltpu.async_copy(
            tabT_hbm.at[:, pl.ds(pl.multiple_of(p * 128, 128), 128)],
            pans[b], isem)

    def start_out(p, b):
        pltpu.async_copy(
            srows[b], tr_hbm.at[pl.ds(pl.multiple_of(p * 32, 32), 32)], osem)

    def drain_in(b):
        pltpu.make_async_copy(
            tabT_hbm.at[:, pl.ds(0, 128)], pans[b], isem).wait()

    def drain_out(b):
        pltpu.make_async_copy(
            srows[b], tr_hbm.at[pl.ds(0, 32)], osem).wait()

    @pl.when(n > 0)
    def _():
        start_in(p0, 0)
    @pl.when(n > 1)
    def _():
        start_in(p0 + 1, 1)

    def step(i, carry):
        for b in range(2):
            p = p0 + i * 2 + b

            @pl.when(p < phi)
            def _body():
                drain_in(b)
                @pl.when(i > 0)
                def _reclaim():
                    drain_out(b)
                _transpose_panel(pans[b], srows[b], 32)
                start_out(p, b)

                @pl.when(p + 2 < phi)
                def _next():
                    start_in(p + 2, b)
        return carry

    lax.fori_loop(0, (n + 1) // 2, step, 0)

    @pl.when(n > 0)
    def _d0():
        drain_out(0)
    @pl.when(n > 1)
    def _d1():
        drain_out(1)

    @pl.when(wid == NW - 1)
    def _():
        pltpu.sync_copy(tail_hbm, tr_hbm.at[pl.ds(PFULL * 32, 16)])


def _body_b(tr_hbm, x_hbm, sh_hbm, off_hbm, out_hbm,
            idx_v, sidx_v, qoff_v, srows_v, o_v, sh_v, shx_v, off_v, sem):
    wid = lax.axis_index("s") * NC + lax.axis_index("c")
    base = wid * PER_W

    pltpu.sync_copy(sh_hbm, sh_v)
    pltpu.sync_copy(off_hbm, off_v)

    iota = lax.iota(jnp.int32, 16)
    r2pat = lax.shift_right_logical(iota, 2)
    c2pat = lax.shift_left(jnp.bitwise_and(iota, 3), 5)

    for phase in range(13):
        fv = iota + (phase * 16) % F
        fv = jnp.where(fv >= F, fv - F, fv)
        fb = fv * D
        for dd in range(D):
            shx_v[pl.ds((phase * D + dd) * 16, 16)] = plsc.load_gather(
                sh_v, [fb + dd])

    def chunk(c, _):
        start = pl.multiple_of(base + c * CH, CH)
        pltpu.sync_copy(x_hbm.at[pl.ds(start, CH)], idx_v)
        for k in range(GPC):
            sl = pl.ds(k * 16, 16)
            t = idx_v[sl] + off_v[sl]
            sidx_v[sl] = lax.shift_right_logical(t, 2)
            qoff_v[sl] = lax.shift_left(jnp.bitwise_and(t, 3), 5)

        copies = []
        for j in range(NSTREAM):
            copies.append(pltpu.async_copy(
                tr_hbm.at[sidx_v.at[pl.ds(j * 104, 104)]],
                srows_v.at[pl.ds(j * 104, 104)],
                sem))
        for cp in copies:
            cp.wait()

        def half(h, _):
            rb = h * 208
            for g in range(13):
                rowbase = rb + g * 16
                rvec = rowbase + iota
                qv = plsc.load_gather(qoff_v, [rvec])
                r2 = (rowbase // 4) + r2pat
                for dd in range(D):
                    val = plsc.load_gather(srows_v, [rvec, qv + dd])
                    val = val + shx_v[pl.ds((g * D + dd) * 16, 16)]
                    plsc.store_scatter(o_v, [r2, c2pat + dd], val)
            return _
        lax.fori_loop(0, 2, half, 0)

        pltpu.sync_copy(
            o_v, out_hbm.at[pl.ds(pl.multiple_of(start // 4, CH // 4), CH // 4)])
        return _

    lax.fori_loop(0, NCHUNK, chunk, 0)


def kernel(x, indiv_embed, shared_embed):
    offsets = np.arange(F, dtype=np.int32) * CARD
    offpat = jnp.asarray(np.tile(offsets, CH // F))

    mesh = plsc.VectorSubcoreMesh(core_axis_name="c", subcore_axis_name="s")

    run_a = pl.kernel(
        _body_a,
        out_type=jax.ShapeDtypeStruct((NSR, 128), jnp.float32),
        mesh=mesh,
        scratch_types=[
            pltpu.VMEM((32, 128), jnp.float32),
            pltpu.VMEM((32, 128), jnp.float32),
            pltpu.VMEM((32, 128), jnp.float32),
            pltpu.VMEM((32, 128), jnp.float32),
            pltpu.SemaphoreType.DMA,
            pltpu.SemaphoreType.DMA,
        ],
        compiler_params=_cparams,
    )

    run_b = pl.kernel(
        _body_b,
        out_type=jax.ShapeDtypeStruct((BF // 4, 128), jnp.float32),
        mesh=mesh,
        scratch_types=[
            pltpu.VMEM((CH,), jnp.int32),
            pltpu.VMEM((CH,), jnp.int32),
            pltpu.VMEM((CH,), jnp.int32),
            pltpu.VMEM((CH, 128), jnp.float32),
            pltpu.VMEM((CH // 4, 128), jnp.float32),
            pltpu.VMEM((F * D,), jnp.float32),
            pltpu.VMEM((13 * D * 16,), jnp.float32),
            pltpu.VMEM((CH,), jnp.int32),
            pltpu.SemaphoreType.DMA,
        ],
        compiler_params=_cparams,
    )

    tail = indiv_embed[V - 64:].reshape(16, 128)
    tr = run_a(indiv_embed.T, tail)
    out = run_b(tr, x.reshape(BF), shared_embed.reshape(F * D), offpat)
    return out.reshape(B, F, D)


# speedup vs baseline: 2.0990x; 1.4365x over previous
import functools

import jax
import jax.numpy as jnp
import numpy as np
from jax import lax
from jax.experimental import pallas as pl
from jax.experimental.pallas import tpu as pltpu
from jax.experimental.pallas import tpu_sc as plsc

B = 16384
F = 26
D = 32
BF = B * F
CARD = 100000

NC = 2
NS = 16
NW = NC * NS
PER_W = BF // NW

R = 1664
NCHUNK = PER_W // R
NSTREAM = R // 128
GROUPS = R // F


def _body(x_hbm, shared_hbm, offpat_hbm, table_hbm, out_hbm,
          idx_v, rows_v, shared_v, offpat_v, sem):
    wid = lax.axis_index("s") * NC + lax.axis_index("c")
    base = wid * PER_W

    pltpu.sync_copy(shared_hbm, shared_v)
    pltpu.sync_copy(offpat_hbm, offpat_v)

    for c in range(NCHUNK):
        pltpu.sync_copy(x_hbm.at[wid * NCHUNK + c], idx_v)
        for j in range(NSTREAM):
            for k in range(128 // 16):
                sl = pl.ds(k * 16, 16)
                idx_v[j, sl] = idx_v[j, sl] + offpat_v[j, sl]

        copies = []
        for j in range(NSTREAM):
            copies.append(pltpu.async_copy(
                table_hbm.at[idx_v.at[j]],
                rows_v.at[pl.ds(j * 128, 128)],
                sem))
        for cp in copies:
            cp.wait()

        def add_group(g, carry):
            r0 = g * F
            for r in range(F):
                for col in (0, 16):
                    sl = pl.ds(col, 16)
                    rows_v[r0 + r, sl] = rows_v[r0 + r, sl] + shared_v[r, sl]
            return carry
        lax.fori_loop(0, GROUPS, add_group, 0)

        pltpu.sync_copy(rows_v, out_hbm.at[pl.ds(base + c * R, R)])


def kernel(x, indiv_embed, shared_embed):
    offsets = (np.arange(F, dtype=np.int32) * CARD)
    offpat = jnp.asarray(np.tile(offsets, R // F).reshape(NSTREAM, 128))
    x3d = x.reshape(NW * NCHUNK, NSTREAM, 128)

    mesh = plsc.VectorSubcoreMesh(core_axis_name="c", subcore_axis_name="s")
    run = pl.kernel(
        _body,
        out_type=jax.ShapeDtypeStruct((BF, D), jnp.float32),
        mesh=mesh,
        scratch_types=[
            pltpu.VMEM((NSTREAM, 128), jnp.int32),
            pltpu.VMEM((R, D), jnp.float32),
            pltpu.VMEM((F, D), jnp.float32),
            pltpu.VMEM((NSTREAM, 128), jnp.int32),
            pltpu.SemaphoreType.DMA,
        ],
        compiler_params=pltpu.CompilerParams(use_tc_tiling_on_sc=False),
    )
    out = run(x3d, shared_embed, offpat, indiv_embed)
    return out.reshape(B, F, D)


# speedup vs baseline: 2.1263x; 1.0130x over previous
import functools

import jax
import jax.numpy as jnp
import numpy as np
from jax import lax
from jax.experimental import pallas as pl
from jax.experimental.pallas import tpu as pltpu
from jax.experimental.pallas import tpu_sc as plsc

B = 16384
F = 26
D = 32
BF = B * F
CARD = 100000

NC = 2
NS = 16
NW = NC * NS
PER_W = BF // NW

R = 1664
NCHUNK = PER_W // R
NSTREAM = R // 128
GROUPS = R // F


def _body(x_hbm, shared_hbm, offpat_hbm, table_hbm, out_hbm,
          idx0_v, idx1_v, rows0_v, rows1_v, shared_v, offpat_v, gsem, osem):
    wid = lax.axis_index("s") * NC + lax.axis_index("c")
    base = wid * PER_W
    idxs = (idx0_v, idx1_v)
    rows = (rows0_v, rows1_v)

    pltpu.sync_copy(shared_hbm, shared_v)
    pltpu.sync_copy(offpat_hbm, offpat_v)

    def fire_gathers(c, b):
        pltpu.sync_copy(x_hbm.at[wid * NCHUNK + c], idxs[b])
        for j in range(NSTREAM):
            for k in range(128 // 16):
                sl = pl.ds(k * 16, 16)
                idxs[b][j, sl] = idxs[b][j, sl] + offpat_v[j, sl]
        for j in range(NSTREAM):
            pltpu.async_copy(
                table_hbm.at[idxs[b].at[j]],
                rows[b].at[pl.ds(j * 128, 128)],
                gsem)

    def drain_gathers(b):
        for j in range(NSTREAM):
            pltpu.make_async_copy(
                table_hbm.at[idxs[b].at[j]],
                rows[b].at[pl.ds(j * 128, 128)],
                gsem).wait()

    def drain_out(c, b):
        pltpu.make_async_copy(
            rows[b], out_hbm.at[pl.ds(base + c * R, R)], osem).wait()

    fire_gathers(0, 0)
    for c in range(NCHUNK):
        b = c % 2
        if c + 1 < NCHUNK:
            if c >= 1:
                drain_out(c - 1, 1 - b)
            fire_gathers(c + 1, 1 - b)
        drain_gathers(b)

        def add_group(g, carry):
            r0 = g * F
            for r in range(F):
                for col in (0, 16):
                    sl = pl.ds(col, 16)
                    rows[b][r0 + r, sl] = rows[b][r0 + r, sl] + shared_v[r, sl]
            return carry
        lax.fori_loop(0, GROUPS, add_group, 0)

        pltpu.async_copy(rows[b], out_hbm.at[pl.ds(base + c * R, R)], osem)

    drain_out(NCHUNK - 2, 0)
    drain_out(NCHUNK - 1, 1)


def kernel(x, indiv_embed, shared_embed):
    offsets = (np.arange(F, dtype=np.int32) * CARD)
    offpat = jnp.asarray(np.tile(offsets, R // F).reshape(NSTREAM, 128))
    x3d = x.reshape(NW * NCHUNK, NSTREAM, 128)

    mesh = plsc.VectorSubcoreMesh(core_axis_name="c", subcore_axis_name="s")
    run = pl.kernel(
        _body,
        out_type=jax.ShapeDtypeStruct((BF, D), jnp.float32),
        mesh=mesh,
        scratch_types=[
            pltpu.VMEM((NSTREAM, 128), jnp.int32),
            pltpu.VMEM((NSTREAM, 128), jnp.int32),
            pltpu.VMEM((R, D), jnp.float32),
            pltpu.VMEM((R, D), jnp.float32),
            pltpu.VMEM((F, D), jnp.float32),
            pltpu.VMEM((NSTREAM, 128), jnp.int32),
            pltpu.SemaphoreType.DMA,
            pltpu.SemaphoreType.DMA,
        ],
        compiler_params=pltpu.CompilerParams(use_tc_tiling_on_sc=False),
    )
    out = run(x3d, shared_embed, offpat, indiv_embed)
    return out.reshape(B, F, D)
